# baseline (speedup 1.0000x reference)
import functools

import jax
import jax.numpy as jnp
from jax import lax
from jax.experimental import pallas as pl
from jax.experimental.pallas import tpu as pltpu
from jax.experimental.pallas import tpu_sc as plsc

EMBED_D = 64
NUM_CORES = 2
NUM_SUBCORES = 16
NUM_WORKERS = NUM_CORES * NUM_SUBCORES
CHUNK = 128
K = 5
GROUP = CHUNK * K


def _make_lookup(total_rows: int):
  rows_per_w = total_rows // NUM_WORKERS
  idx_rows_per_w = rows_per_w // CHUNK
  n_groups = idx_rows_per_w // K
  assert rows_per_w % (CHUNK * K) == 0 and n_groups >= 2

  mesh = plsc.VectorSubcoreMesh(
      core_axis_name="c", subcore_axis_name="s", num_cores=NUM_CORES)

  @functools.partial(
      pl.kernel,
      out_type=jax.ShapeDtypeStruct((total_rows, EMBED_D), jnp.float32),
      mesh=mesh,
      compiler_params=pltpu.CompilerParams(use_tc_tiling_on_sc=False),
      scratch_types=[
          pltpu.VMEM((idx_rows_per_w, CHUNK), jnp.int32),
          pltpu.VMEM((GROUP, EMBED_D), jnp.float32),
          pltpu.VMEM((GROUP, EMBED_D), jnp.float32),
          pltpu.SemaphoreType.DMA,
          pltpu.SemaphoreType.DMA,
          pltpu.SemaphoreType.DMA,
          pltpu.SemaphoreType.DMA,
      ],
  )
  def lookup(table_hbm, idx_hbm, out_hbm, idx_v, rows0, rows1, gsem0, gsem1,
             ssem0, ssem1):
    wid = lax.axis_index("s") * NUM_CORES + lax.axis_index("c")
    idx_base = wid * idx_rows_per_w
    out_base = wid * rows_per_w

    pltpu.sync_copy(idx_hbm.at[pl.ds(idx_base, idx_rows_per_w)], idx_v)

    bufs = ((rows0, gsem0, ssem0), (rows1, gsem1, ssem1))

    def fire_gathers(g, rows_v, gsem):
      for j in range(K):
        pltpu.async_copy(
            table_hbm.at[idx_v.at[g * K + j]],
            rows_v.at[pl.ds(j * CHUNK, CHUNK)], gsem)

    def wait_gathers(g, rows_v, gsem):
      for j in range(K):
        pltpu.make_async_copy(
            table_hbm.at[idx_v.at[g * K + j]],
            rows_v.at[pl.ds(j * CHUNK, CHUNK)], gsem).wait()

    def out_slice(g):
      return out_hbm.at[pl.ds(out_base + g * GROUP, GROUP)]

    fire_gathers(0, rows0, gsem0)
    for g in range(n_groups):
      rows_v, gsem, ssem = bufs[g % 2]
      if g + 1 < n_groups:
        rows_n, gsem_n, ssem_n = bufs[(g + 1) % 2]
        if g >= 1:
          pltpu.make_async_copy(rows_n, out_slice(g - 1), ssem_n).wait()
        fire_gathers(g + 1, rows_n, gsem_n)
      wait_gathers(g, rows_v, gsem)
      pltpu.async_copy(rows_v, out_slice(g), ssem)

    r2, _, s2 = bufs[(n_groups - 2) % 2]
    r1, _, s1 = bufs[(n_groups - 1) % 2]
    pltpu.make_async_copy(r2, out_slice(n_groups - 2), s2).wait()
    pltpu.make_async_copy(r1, out_slice(n_groups - 1), s1).wait()

  return lookup


def kernel(ids, W):
  flat_ids = ids.reshape(-1).astype(jnp.int32)
  total_rows = flat_ids.shape[0]
  idx2d = flat_ids.reshape(total_rows // CHUNK, CHUNK)
  out = _make_lookup(total_rows)(W, idx2d)
  out = out.reshape(ids.shape + (EMBED_D,))
  zero = lax.optimization_barrier(jnp.zeros((), jnp.float32))
  return out + zero


# speedup vs baseline: 1.5519x; 1.5519x over previous
import functools

import jax
import jax.numpy as jnp
from jax import lax
from jax.experimental import pallas as pl
from jax.experimental.pallas import tpu as pltpu
from jax.experimental.pallas import tpu_sc as plsc

EMBED_D = 64
NUM_CORES = 2
NUM_SUBCORES = 16
NUM_WORKERS = NUM_CORES * NUM_SUBCORES
CHUNK = 128
NBUF = 4


def _make_lookup(total_rows: int):
  packed_rows = total_rows // 2
  rows_per_w = packed_rows // NUM_WORKERS
  n_groups = rows_per_w // CHUNK
  idx_rows_per_w = n_groups

  mesh = plsc.VectorSubcoreMesh(
      core_axis_name="c", subcore_axis_name="s", num_cores=NUM_CORES)

  @functools.partial(
      pl.kernel,
      out_type=jax.ShapeDtypeStruct((packed_rows, 2 * EMBED_D), jnp.float32),
      mesh=mesh,
      compiler_params=pltpu.CompilerParams(use_tc_tiling_on_sc=False),
      scratch_types=[
          pltpu.VMEM((2 * idx_rows_per_w, CHUNK), jnp.int32),
          pltpu.VMEM((CHUNK, EMBED_D), jnp.float32),
          pltpu.VMEM((CHUNK, EMBED_D), jnp.float32),
          pltpu.VMEM((CHUNK, EMBED_D), jnp.float32),
          pltpu.VMEM((CHUNK, EMBED_D), jnp.float32),
          pltpu.VMEM((CHUNK, EMBED_D), jnp.float32),
          pltpu.VMEM((CHUNK, EMBED_D), jnp.float32),
          pltpu.VMEM((CHUNK, EMBED_D), jnp.float32),
          pltpu.VMEM((CHUNK, EMBED_D), jnp.float32),
          pltpu.SemaphoreType.DMA,
          pltpu.SemaphoreType.DMA,
          pltpu.SemaphoreType.DMA,
          pltpu.SemaphoreType.DMA,
          pltpu.SemaphoreType.DMA,
          pltpu.SemaphoreType.DMA,
          pltpu.SemaphoreType.DMA,
          pltpu.SemaphoreType.DMA,
      ],
  )
  def lookup(table_hbm, idx_even_hbm, idx_odd_hbm, out_hbm, idx_v,
             e0, o0, e1, o1, e2, o2, e3, o3,
             g0, g1, g2, g3, s0, s1, s2, s3):
    wid = lax.axis_index("s") * NUM_CORES + lax.axis_index("c")
    idx_base = wid * idx_rows_per_w
    out_base = wid * rows_per_w

    pltpu.sync_copy(idx_even_hbm.at[pl.ds(idx_base, idx_rows_per_w)],
                    idx_v.at[pl.ds(0, idx_rows_per_w)])
    pltpu.sync_copy(idx_odd_hbm.at[pl.ds(idx_base, idx_rows_per_w)],
                    idx_v.at[pl.ds(idx_rows_per_w, idx_rows_per_w)])

    bufs = ((e0, o0, g0, s0), (e1, o1, g1, s1), (e2, o2, g2, s2),
            (e3, o3, g3, s3))

    def fire_gathers(g):
      be, bo, gsem, _ = bufs[g % NBUF]
      pltpu.async_copy(table_hbm.at[idx_v.at[g]], be, gsem)
      pltpu.async_copy(table_hbm.at[idx_v.at[idx_rows_per_w + g]], bo, gsem)

    def wait_gathers(g):
      be, bo, gsem, _ = bufs[g % NBUF]
      pltpu.make_async_copy(table_hbm.at[idx_v.at[g]], be, gsem).wait()
      pltpu.make_async_copy(table_hbm.at[idx_v.at[idx_rows_per_w + g]], bo,
                            gsem).wait()

    def out_slices(g):
      rows = out_hbm.at[pl.ds(out_base + g * CHUNK, CHUNK)]
      return rows.at[:, pl.ds(0, EMBED_D)], rows.at[:, pl.ds(EMBED_D, EMBED_D)]

    def fire_store(g):
      be, bo, _, ssem = bufs[g % NBUF]
      de, do = out_slices(g)
      pltpu.async_copy(be, de, ssem)
      pltpu.async_copy(bo, do, ssem)

    def wait_store(g):
      be, bo, _, ssem = bufs[g % NBUF]
      de, do = out_slices(g)
      pltpu.make_async_copy(be, de, ssem).wait()
      pltpu.make_async_copy(bo, do, ssem).wait()

    for g in range(min(NBUF - 1, n_groups)):
      fire_gathers(g)
    for g in range(n_groups):
      nxt = g + NBUF - 1
      if nxt < n_groups:
        if g >= 1:
          wait_store(g - 1)
        fire_gathers(nxt)
      wait_gathers(g)
      fire_store(g)

    for g in range(max(0, n_groups - NBUF + 1), n_groups):
      wait_store(g)

  return lookup


def kernel(ids, W):
  flat_ids = ids.reshape(-1).astype(jnp.int32)
  total_rows = flat_ids.shape[0]
  idx_even = flat_ids[0::2].reshape(total_rows // (2 * CHUNK), CHUNK)
  idx_odd = flat_ids[1::2].reshape(total_rows // (2 * CHUNK), CHUNK)
  out = _make_lookup(total_rows)(W, idx_even, idx_odd)
  return out.reshape(ids.shape + (EMBED_D,))
